# baseline (speedup 1.0000x reference)
import functools

import jax
import jax.numpy as jnp
from jax import lax
from jax.experimental import pallas as pl
from jax.experimental.pallas import tpu as pltpu
from jax.experimental.pallas import tpu_sc as plsc

_LANES = 16
_NUM_WORKERS = 32
_ROWS_PER_CHUNK = 32
_NBUF = 2


def _magic_div(p, d):
    m = -(-(1 << 20) // d)
    return (p * m) >> 20


@jax.jit
def _sc_lookup(vocab_map, tokens):
    n_rows, seq = tokens.shape
    rows_per_worker = n_rows // _NUM_WORKERS
    n_chunks = rows_per_worker // _ROWS_PER_CHUNK
    chunk_elems = _ROWS_PER_CHUNK * seq
    mesh = plsc.VectorSubcoreMesh(
        core_axis_name="c", subcore_axis_name="s", num_cores=2, num_subcores=16
    )

    @functools.partial(
        pl.kernel,
        out_type=jax.ShapeDtypeStruct(tokens.shape, jnp.float32),
        mesh=mesh,
        scratch_types=[
            pltpu.VMEM(vocab_map.shape, jnp.float32),
            [pltpu.VMEM((_ROWS_PER_CHUNK, seq), jnp.int32) for _ in range(_NBUF)],
            [pltpu.VMEM((_ROWS_PER_CHUNK, seq), jnp.float32) for _ in range(_NBUF)],
            pltpu.SemaphoreType.DMA,
            [pltpu.SemaphoreType.DMA for _ in range(_NBUF)],
            [pltpu.SemaphoreType.DMA for _ in range(_NBUF)],
        ],
        compiler_params=pltpu.CompilerParams(
            use_tc_tiling_on_sc=False, needs_layout_passes=False
        ),
    )
    def body(table_hbm, tok_hbm, out_hbm, table_v, idx_v, out_v,
             sem_tab, sem_in, sem_out):
        wid = lax.axis_index("s") * 2 + lax.axis_index("c")
        base = wid * rows_per_worker

        cp_tab = pltpu.async_copy(table_hbm, table_v, sem_tab)
        in_cps = [None] * _NBUF
        out_cps = [None] * _NBUF
        for c in range(min(_NBUF, n_chunks)):
            in_cps[c] = pltpu.async_copy(
                tok_hbm.at[pl.ds(base + c * _ROWS_PER_CHUNK, _ROWS_PER_CHUNK), :],
                idx_v[c], sem_in[c],
            )
        cp_tab.wait()

        lane_iota = lax.iota(jnp.int32, _LANES)

        for c in range(n_chunks):
            b = c % _NBUF
            in_cps[b].wait()
            if out_cps[b] is not None:
                out_cps[b].wait()

            @plsc.parallel_loop(0, chunk_elems, step=_LANES, unroll=8)
            def _(i):
                p = i + lane_iota
                row = _magic_div(p, seq)
                col = p - row * seq
                toks = plsc.load_gather(idx_v[b], [row, col])
                vals = plsc.load_gather(table_v, [toks])
                plsc.store_scatter(out_v[b], [row, col], vals)

            out_cps[b] = pltpu.async_copy(
                out_v[b],
                out_hbm.at[pl.ds(base + c * _ROWS_PER_CHUNK, _ROWS_PER_CHUNK), :],
                sem_out[b],
            )
            nxt = c + _NBUF
            if nxt < n_chunks:
                in_cps[b] = pltpu.async_copy(
                    tok_hbm.at[pl.ds(base + nxt * _ROWS_PER_CHUNK, _ROWS_PER_CHUNK), :],
                    idx_v[b], sem_in[b],
                )
        for b in range(min(_NBUF, n_chunks)):
            if out_cps[b] is not None:
                out_cps[b].wait()

    return body(vocab_map, tokens)


def kernel(tokens, start_idxs, end_idxs, vocab_map):
    return _sc_lookup(vocab_map, tokens), start_idxs, end_idxs
